# Initial kernel scaffold; baseline (speedup 1.0000x reference)
import functools

import jax
import jax.numpy as jnp
from jax import lax
from jax.experimental import pallas as pl
from jax.experimental.pallas import tpu as pltpu
from jax.experimental.pallas import tpu_sc as plsc

D = 64
NC = 2
NS = 16
NW = NC * NS
GRP = 128
GPC = 4
CHUNK = GRP * GPC


@jax.jit
def _sc_gather(table, idx3d):
    n_grp = idx3d.shape[1]
    b_per_w = n_grp * GRP
    n_chunks = n_grp // GPC
    n_rows = NW * b_per_w
    mesh = plsc.VectorSubcoreMesh(core_axis_name="c", subcore_axis_name="s")

    @functools.partial(
        pl.kernel,
        mesh=mesh,
        out_type=jax.ShapeDtypeStruct((n_rows, D), jnp.float32),
        scratch_types=[
            pltpu.VMEM((n_grp, GRP), jnp.int32),
            pltpu.VMEM((CHUNK, D), jnp.float32),
            pltpu.SemaphoreType.DMA,
        ],
    )
    def k(table_hbm, idx_hbm, out_hbm, idx_v, buf, gsem):
        wid = lax.axis_index("s") * NC + lax.axis_index("c")
        base = wid * b_per_w
        pltpu.sync_copy(idx_hbm.at[wid], idx_v)

        def body(c, carry):
            for g in range(GPC):
                pltpu.async_copy(
                    table_hbm.at[idx_v.at[c * GPC + g]],
                    buf.at[pl.ds(g * GRP, GRP)],
                    gsem,
                )
            pltpu.make_async_copy(
                table_hbm.at[pl.ds(0, CHUNK)], buf, gsem
            ).wait()
            pltpu.sync_copy(buf, out_hbm.at[pl.ds(base + c * CHUNK, CHUNK)])
            return carry

        lax.fori_loop(0, n_chunks, body, 0)

    return k(table, idx3d)


def kernel(input_items, embeddings):
    bsz, hist = input_items.shape
    flat = input_items.reshape(-1).astype(jnp.int32)
    n = flat.shape[0]
    idx3d = flat.reshape(NW, n // (NW * GRP), GRP)
    out = _sc_gather(embeddings, idx3d)
    return out.reshape(bsz, hist, D)


# speedup vs baseline: 1.8300x; 1.8300x over previous
import functools

import jax
import jax.numpy as jnp
from jax import lax
from jax.experimental import pallas as pl
from jax.experimental.pallas import tpu as pltpu
from jax.experimental.pallas import tpu_sc as plsc

D = 64
NC = 2
NS = 16
NW = NC * NS
GRP = 128
GPC = 4
CHUNK = GRP * GPC


@jax.jit
def _sc_gather(table, idx3d):
    n_grp = idx3d.shape[1]
    b_per_w = n_grp * GRP
    n_chunks = n_grp // GPC
    n_rows = NW * b_per_w
    mesh = plsc.VectorSubcoreMesh(core_axis_name="c", subcore_axis_name="s")

    @functools.partial(
        pl.kernel,
        mesh=mesh,
        out_type=jax.ShapeDtypeStruct((n_rows, D), jnp.float32),
        scratch_types=[
            pltpu.VMEM((n_grp, GRP), jnp.int32),
            pltpu.VMEM((CHUNK, D), jnp.float32),
            pltpu.SemaphoreType.DMA,
        ],
        compiler_params=pltpu.CompilerParams(use_tc_tiling_on_sc=False),
    )
    def k(table_hbm, idx_hbm, out_hbm, idx_v, buf, gsem):
        wid = lax.axis_index("s") * NC + lax.axis_index("c")
        base = wid * b_per_w
        pltpu.sync_copy(idx_hbm.at[wid], idx_v)

        def body(c, carry):
            for g in range(GPC):
                pltpu.async_copy(
                    table_hbm.at[idx_v.at[c * GPC + g]],
                    buf.at[pl.ds(g * GRP, GRP)],
                    gsem,
                )
            pltpu.make_async_copy(
                table_hbm.at[pl.ds(0, CHUNK)], buf, gsem
            ).wait()
            pltpu.sync_copy(buf, out_hbm.at[pl.ds(base + c * CHUNK, CHUNK)])
            return carry

        lax.fori_loop(0, n_chunks, body, 0)

    return k(table, idx3d)


def kernel(input_items, embeddings):
    bsz, hist = input_items.shape
    flat = input_items.reshape(-1).astype(jnp.int32)
    n = flat.shape[0]
    idx3d = flat.reshape(NW, n // (NW * GRP), GRP)
    out = _sc_gather(embeddings, idx3d)
    return out.reshape(bsz, hist, D)


# speedup vs baseline: 1.8649x; 1.0191x over previous
import functools

import jax
import jax.numpy as jnp
from jax import lax
from jax.experimental import pallas as pl
from jax.experimental.pallas import tpu as pltpu
from jax.experimental.pallas import tpu_sc as plsc

D = 64
NC = 2
NS = 16
NW = NC * NS
GRP = 128
GPC = 4
CHUNK = GRP * GPC


@jax.jit
def _sc_gather(table, idx3d):
    n_grp = idx3d.shape[1]
    b_per_w = n_grp * GRP
    n_chunks = n_grp // GPC
    n_rows = NW * b_per_w
    mesh = plsc.VectorSubcoreMesh(core_axis_name="c", subcore_axis_name="s")

    n_pairs = n_chunks // 2

    @functools.partial(
        pl.kernel,
        mesh=mesh,
        out_type=jax.ShapeDtypeStruct((n_rows, D), jnp.float32),
        scratch_types=[
            pltpu.VMEM((n_grp, GRP), jnp.int32),
            pltpu.VMEM((CHUNK, D), jnp.float32),
            pltpu.VMEM((CHUNK, D), jnp.float32),
            pltpu.SemaphoreType.DMA,
            pltpu.SemaphoreType.DMA,
            pltpu.SemaphoreType.DMA,
            pltpu.SemaphoreType.DMA,
        ],
        compiler_params=pltpu.CompilerParams(use_tc_tiling_on_sc=False),
    )
    def k(table_hbm, idx_hbm, out_hbm, idx_v, buf0, buf1, g0, g1, o0, o1):
        wid = lax.axis_index("s") * NC + lax.axis_index("c")
        base = wid * b_per_w
        pltpu.sync_copy(idx_hbm.at[wid], idx_v)

        def fill(c, buf, sem):
            for g in range(GPC):
                pltpu.async_copy(
                    table_hbm.at[idx_v.at[c * GPC + g]],
                    buf.at[pl.ds(g * GRP, GRP)],
                    sem,
                )

        def drain_fill(buf, sem):
            pltpu.make_async_copy(
                table_hbm.at[pl.ds(0, CHUNK)], buf, sem
            ).wait()

        def writeout(buf, c, sem):
            pltpu.async_copy(buf, out_hbm.at[pl.ds(base + c * CHUNK, CHUNK)], sem)

        def drain_writeout(buf, sem):
            pltpu.make_async_copy(
                buf, out_hbm.at[pl.ds(base, CHUNK)], sem
            ).wait()

        fill(0, buf0, g0)
        fill(1, buf1, g1)

        def body(p, carry):
            c0 = 2 * p
            drain_fill(buf0, g0)
            writeout(buf0, c0, o0)
            drain_fill(buf1, g1)
            writeout(buf1, c0 + 1, o1)
            drain_writeout(buf0, o0)
            fill(c0 + 2, buf0, g0)
            drain_writeout(buf1, o1)
            fill(c0 + 3, buf1, g1)
            return carry

        lax.fori_loop(0, n_pairs - 1, body, 0)

        c0 = 2 * (n_pairs - 1)
        drain_fill(buf0, g0)
        writeout(buf0, c0, o0)
        drain_fill(buf1, g1)
        writeout(buf1, c0 + 1, o1)
        drain_writeout(buf0, o0)
        drain_writeout(buf1, o1)

    return k(table, idx3d)


def kernel(input_items, embeddings):
    bsz, hist = input_items.shape
    flat = input_items.reshape(-1).astype(jnp.int32)
    n = flat.shape[0]
    idx3d = flat.reshape(NW, n // (NW * GRP), GRP)
    out = _sc_gather(embeddings, idx3d)
    return out.reshape(bsz, hist, D)
